# baseline (speedup 1.0000x reference)
import functools

import jax
import jax.numpy as jnp
from jax import lax
from jax.experimental import pallas as pl
from jax.experimental.pallas import tpu as pltpu
from jax.experimental.pallas import tpu_sc as plsc

N = 10000
E = 80000
D = 512
NCH = 4
CW = D // NCH
NS = 16
EPT = E // NS
BE = 50
NB = EPT // BE
NA = 10240
RPT = NA // NS
ZR = 32

HT = 8
HEPT = E // HT
NH = 10240
HS = NH // NS

_f32 = jnp.float32
_i32 = jnp.int32


def _sc_mesh():
    return plsc.VectorSubcoreMesh(core_axis_name="c", subcore_axis_name="s",
                                  num_cores=2, num_subcores=NS)


_SC_PARAMS = pltpu.CompilerParams(needs_layout_passes=False)


def _sc_degrees(ia, ib, ic, id_):
    out_t = tuple(jax.ShapeDtypeStruct((NH,), _f32) for _ in range(4))

    @functools.partial(
        pl.kernel, out_type=out_t, mesh=_sc_mesh(),
        compiler_params=_SC_PARAMS,
        scratch_types=dict(
            idxv=pltpu.VMEM((HEPT,), _i32),
            hist=pltpu.VMEM((NH,), _f32),
            red=pltpu.VMEM((HT, HS), _f32),
            res=pltpu.VMEM((HS,), _f32),
            shist=pltpu.VMEM_SHARED((2, HT, NH), _f32),
        ),
    )
    def k(a_hbm, b_hbm, c_hbm, d_hbm, oa, ob, oc, od,
          idxv, hist, red, res, shist):
        c = lax.axis_index("c")
        s = lax.axis_index("s")
        grp = s // HT
        row = s - grp * HT

        for cc, g, ref in ((0, 0, a_hbm), (0, 1, b_hbm),
                           (1, 0, c_hbm), (1, 1, d_hbm)):
            @pl.when(jnp.logical_and(c == cc, grp == g))
            def _(ref=ref):
                pltpu.sync_copy(ref.at[row], idxv)

        zero16 = jnp.zeros((16,), _f32)

        @pl.loop(0, NH, step=16)
        def _(i):
            hist[pl.ds(i, 16)] = zero16

        ones16 = jnp.ones((16,), _f32)

        @pl.loop(0, HEPT, step=16)
        def _(i):
            idx16 = idxv[pl.ds(i, 16)]
            plsc.addupdate_scatter(hist, [idx16], ones16)

        pltpu.sync_copy(hist, shist.at[grp, row])
        plsc.subcore_barrier()

        for g in range(2):
            pltpu.sync_copy(shist.at[g, :, pl.ds(s * HS, HS)], red)

            @pl.loop(0, HS, step=16)
            def _(i):
                acc = red[0, pl.ds(i, 16)]
                for r in range(1, HT):
                    acc = acc + red[r, pl.ds(i, 16)]
                res[pl.ds(i, 16)] = acc

            for cc, out in ((0, (oa, ob)[g]), (1, (oc, od)[g])):
                @pl.when(c == cc)
                def _(out=out):
                    pltpu.sync_copy(res, out.at[pl.ds(s * HS, HS)])

    return k(ia, ib, ic, id_)


def _edge_pass(h, srcv, dstv, g0, g1, acc, sem0, sem1, sem2, sem3):
    pltpu.async_copy(h.at[srcv.at[0]], g0, sem0)
    pltpu.async_copy(h.at[srcv.at[1]], g1, sem1)

    @pl.loop(0, NB, step=2)
    def _(j):
        pltpu.make_async_copy(h.at[srcv.at[j]], g0, sem0).wait()
        pltpu.async_copy(g0, acc.at[dstv.at[j]], sem2, add=True)
        pltpu.make_async_copy(h.at[srcv.at[j + 1]], g1, sem1).wait()
        pltpu.async_copy(g1, acc.at[dstv.at[j + 1]], sem3, add=True)

        @pl.when(j + 2 < NB)
        def _():
            pltpu.make_async_copy(g0, acc.at[dstv.at[j]], sem2).wait()
            pltpu.async_copy(h.at[srcv.at[j + 2]], g0, sem0)
            pltpu.make_async_copy(g1, acc.at[dstv.at[j + 1]], sem3).wait()
            pltpu.async_copy(h.at[srcv.at[j + 3]], g1, sem1)

    pltpu.make_async_copy(g0, acc.at[dstv.at[NB - 2]], sem2).wait()
    pltpu.make_async_copy(g1, acc.at[dstv.at[NB - 1]], sem3).wait()


def _sc_aggregate(h0, h1, h2, h3, src_g, dst_g):
    out_t = tuple(jax.ShapeDtypeStruct((NA, CW), _f32) for _ in range(4))

    @functools.partial(
        pl.kernel, out_type=out_t, mesh=_sc_mesh(),
        compiler_params=_SC_PARAMS,
        scratch_types=dict(
            srcv=pltpu.VMEM((NB, BE), _i32),
            dstv=pltpu.VMEM((NB, BE), _i32),
            g0=pltpu.VMEM((BE, CW), _f32),
            g1=pltpu.VMEM((BE, CW), _f32),
            zb=pltpu.VMEM((ZR, CW), _f32),
            acc=pltpu.VMEM_SHARED((NA, CW), _f32),
            sem0=pltpu.SemaphoreType.DMA,
            sem1=pltpu.SemaphoreType.DMA,
            sem2=pltpu.SemaphoreType.DMA,
            sem3=pltpu.SemaphoreType.DMA,
        ),
    )
    def k(h0_hbm, h1_hbm, h2_hbm, h3_hbm, src_hbm, dst_hbm,
          o0, o1, o2, o3, srcv, dstv, g0, g1, zb, acc,
          sem0, sem1, sem2, sem3):
        c = lax.axis_index("c")
        s = lax.axis_index("s")
        pltpu.sync_copy(src_hbm.at[s], srcv)
        pltpu.sync_copy(dst_hbm.at[s], dstv)

        zero16 = jnp.zeros((16,), _f32)

        @pl.loop(0, ZR)
        def _(i):
            for j in range(CW // 16):
                zb[i, pl.ds(j * 16, 16)] = zero16

        hs = (h0_hbm, h1_hbm, h2_hbm, h3_hbm)
        outs = (o0, o1, o2, o3)
        for it in range(2):
            for z in range(RPT // ZR):
                pltpu.sync_copy(zb, acc.at[pl.ds(s * RPT + z * ZR, ZR)])
            plsc.subcore_barrier()
            for core in range(2):
                @pl.when(c == core)
                def _(h=hs[core * 2 + it]):
                    _edge_pass(h, srcv, dstv, g0, g1, acc,
                               sem0, sem1, sem2, sem3)
            plsc.subcore_barrier()
            for core in range(2):
                @pl.when(c == core)
                def _(o=outs[core * 2 + it]):
                    pltpu.sync_copy(acc.at[pl.ds(s * RPT, RPT)],
                                    o.at[pl.ds(s * RPT, RPT)])
            plsc.subcore_barrier()

    return k(h0, h1, h2, h3, src_g, dst_g)


_BN = 1000


def _rsqrt_clip(d):
    return lax.rsqrt(jnp.maximum(d, 1.0))


def _tc_project(x, W, b, deg_out):
    def body(x_ref, w_ref, b_ref, d_ref, *o_refs):
        y = jnp.dot(x_ref[...], w_ref[...], preferred_element_type=_f32)
        y = (y + b_ref[...]) * _rsqrt_clip(d_ref[...])
        for ci in range(NCH):
            o_refs[ci][...] = y[:, ci * CW:(ci + 1) * CW]

    return pl.pallas_call(
        body,
        grid=(N // _BN,),
        in_specs=[
            pl.BlockSpec((_BN, D), lambda i: (i, 0)),
            pl.BlockSpec((D, D), lambda i: (0, 0)),
            pl.BlockSpec((1, D), lambda i: (0, 0)),
            pl.BlockSpec((_BN, 1), lambda i: (i, 0)),
        ],
        out_specs=[pl.BlockSpec((_BN, CW), lambda i: (i, 0))] * NCH,
        out_shape=[jax.ShapeDtypeStruct((N, CW), _f32)] * NCH,
    )(x, W, b.reshape(1, D), deg_out)


def _tc_mid(a0, a1, a2, a3, deg_in, W, b, deg_out):
    def body(a0r, a1r, a2r, a3r, di_r, w_ref, b_ref, do_r, *o_refs):
        x = jnp.concatenate([a0r[...], a1r[...], a2r[...], a3r[...]], axis=1)
        x = x * _rsqrt_clip(di_r[...])
        y = jnp.dot(x, w_ref[...], preferred_element_type=_f32) + b_ref[...]
        y = jnp.maximum(y, 0.0) * _rsqrt_clip(do_r[...])
        for ci in range(NCH):
            o_refs[ci][...] = y[:, ci * CW:(ci + 1) * CW]

    return pl.pallas_call(
        body,
        grid=(N // _BN,),
        in_specs=[pl.BlockSpec((_BN, CW), lambda i: (i, 0))] * NCH + [
            pl.BlockSpec((_BN, 1), lambda i: (i, 0)),
            pl.BlockSpec((D, D), lambda i: (0, 0)),
            pl.BlockSpec((1, D), lambda i: (0, 0)),
            pl.BlockSpec((_BN, 1), lambda i: (i, 0)),
        ],
        out_specs=[pl.BlockSpec((_BN, CW), lambda i: (i, 0))] * NCH,
        out_shape=[jax.ShapeDtypeStruct((N, CW), _f32)] * NCH,
    )(a0, a1, a2, a3, deg_in, W, b.reshape(1, D), deg_out)


def _tc_final(a0, a1, a2, a3, deg_in, W, b):
    def body(a0r, a1r, a2r, a3r, di_r, w_ref, b_ref, o_ref):
        x = jnp.concatenate([a0r[...], a1r[...], a2r[...], a3r[...]], axis=1)
        x = x * _rsqrt_clip(di_r[...])
        o_ref[...] = (jnp.dot(x, w_ref[...], preferred_element_type=_f32)
                      + b_ref[...])

    return pl.pallas_call(
        body,
        grid=(N // _BN,),
        in_specs=[pl.BlockSpec((_BN, CW), lambda i: (i, 0))] * NCH + [
            pl.BlockSpec((_BN, 1), lambda i: (i, 0)),
            pl.BlockSpec((D, D), lambda i: (0, 0)),
            pl.BlockSpec((1, D), lambda i: (0, 0)),
        ],
        out_specs=pl.BlockSpec((_BN, D), lambda i: (i, 0)),
        out_shape=jax.ShapeDtypeStruct((N, D), _f32),
    )(a0, a1, a2, a3, deg_in, W, b.reshape(1, D))


def kernel(x_user, x_item, edge_rates, edge_rated_by,
           W_in_user, b_in_user, W_in_item, b_in_item,
           W1_rates, b1_rates, W1_rated_by, b1_rated_by,
           W2_rates, b2_rates, W2_rated_by, b2_rated_by):
    er_src = edge_rates[0].astype(_i32)
    er_dst = edge_rates[1].astype(_i32)
    eb_src = edge_rated_by[0].astype(_i32)
    eb_dst = edge_rated_by[1].astype(_i32)

    hu_out, hi_in, hi_out, hu_in = _sc_degrees(
        er_src.reshape(HT, HEPT), er_dst.reshape(HT, HEPT),
        eb_src.reshape(HT, HEPT), eb_dst.reshape(HT, HEPT))
    du_out = hu_out.reshape(NH, 1)
    di_in = hi_in.reshape(NH, 1)
    di_out = hi_out.reshape(NH, 1)
    du_in = hu_in.reshape(NH, 1)

    srg = er_src.reshape(NS, NB, BE)
    drg = er_dst.reshape(NS, NB, BE)
    srb = eb_src.reshape(NS, NB, BE)
    drb = eb_dst.reshape(NS, NB, BE)

    hu = _tc_project(x_user, W_in_user, b_in_user, du_out)
    hi = _tc_project(x_item, W_in_item, b_in_item, di_out)

    ai1 = _sc_aggregate(*hu, srg, drg)
    au1 = _sc_aggregate(*hi, srb, drb)

    h1i = _tc_mid(*ai1, di_in, W1_rates, b1_rates, di_out)
    h1u = _tc_mid(*au1, du_in, W1_rated_by, b1_rated_by, du_out)

    ai2 = _sc_aggregate(*h1u, srg, drg)
    au2 = _sc_aggregate(*h1i, srb, drb)

    oi = _tc_final(*ai2, di_in, W2_rates, b2_rates)
    ou = _tc_final(*au2, du_in, W2_rated_by, b2_rated_by)
    return jnp.concatenate([ou, oi], axis=0)


# speedup vs baseline: 1.1860x; 1.1860x over previous
import functools

import jax
import jax.numpy as jnp
from jax import lax
from jax.experimental import pallas as pl
from jax.experimental.pallas import tpu as pltpu
from jax.experimental.pallas import tpu_sc as plsc

N = 10000
E = 80000
D = 512
NCH = 4
CW = D // NCH
NS = 16
EPT = E // NS
BE = 50
NB = EPT // BE
NA = 10240
RPT = NA // NS
ZR = 32

HT = 8
HEPT = E // HT
NH = 10240
HS = NH // NS

_f32 = jnp.float32
_i32 = jnp.int32


def _sc_mesh():
    return plsc.VectorSubcoreMesh(core_axis_name="c", subcore_axis_name="s",
                                  num_cores=2, num_subcores=NS)


_SC_PARAMS = pltpu.CompilerParams(needs_layout_passes=False)


def _sc_degrees(ia, ib, ic, id_):
    out_t = tuple(jax.ShapeDtypeStruct((NH,), _f32) for _ in range(4))

    @functools.partial(
        pl.kernel, out_type=out_t, mesh=_sc_mesh(),
        compiler_params=_SC_PARAMS,
        scratch_types=dict(
            idxv=pltpu.VMEM((HEPT,), _i32),
            hist=pltpu.VMEM((NH,), _f32),
            red=pltpu.VMEM((HT, HS), _f32),
            res=pltpu.VMEM((HS,), _f32),
            shist=pltpu.VMEM_SHARED((2, HT, NH), _f32),
        ),
    )
    def k(a_hbm, b_hbm, c_hbm, d_hbm, oa, ob, oc, od,
          idxv, hist, red, res, shist):
        c = lax.axis_index("c")
        s = lax.axis_index("s")
        grp = s // HT
        row = s - grp * HT

        for cc, g, ref in ((0, 0, a_hbm), (0, 1, b_hbm),
                           (1, 0, c_hbm), (1, 1, d_hbm)):
            @pl.when(jnp.logical_and(c == cc, grp == g))
            def _(ref=ref):
                pltpu.sync_copy(ref.at[row], idxv)

        zero16 = jnp.zeros((16,), _f32)

        @pl.loop(0, NH, step=16)
        def _(i):
            hist[pl.ds(i, 16)] = zero16

        ones16 = jnp.ones((16,), _f32)

        @pl.loop(0, HEPT, step=16)
        def _(i):
            idx16 = idxv[pl.ds(i, 16)]
            plsc.addupdate_scatter(hist, [idx16], ones16)

        pltpu.sync_copy(hist, shist.at[grp, row])
        plsc.subcore_barrier()

        for g in range(2):
            pltpu.sync_copy(shist.at[g, :, pl.ds(s * HS, HS)], red)

            @pl.loop(0, HS, step=16)
            def _(i):
                acc = red[0, pl.ds(i, 16)]
                for r in range(1, HT):
                    acc = acc + red[r, pl.ds(i, 16)]
                res[pl.ds(i, 16)] = acc

            for cc, out in ((0, (oa, ob)[g]), (1, (oc, od)[g])):
                @pl.when(c == cc)
                def _(out=out):
                    pltpu.sync_copy(res, out.at[pl.ds(s * HS, HS)])

    return k(ia, ib, ic, id_)


def _edge_pass(h, srcv, dstv, g0, g1, acc, sem0, sem1):
    pltpu.async_copy(h.at[srcv.at[0]], g0, sem0)

    @pl.loop(0, NB, step=2)
    def _(j):
        pltpu.async_copy(h.at[srcv.at[j + 1]], g1, sem1)
        pltpu.make_async_copy(h.at[srcv.at[j]], g0, sem0).wait()
        pltpu.sync_copy(g0, acc.at[dstv.at[j]], add=True)

        @pl.when(j + 2 < NB)
        def _():
            pltpu.async_copy(h.at[srcv.at[j + 2]], g0, sem0)

        pltpu.make_async_copy(h.at[srcv.at[j + 1]], g1, sem1).wait()
        pltpu.sync_copy(g1, acc.at[dstv.at[j + 1]], add=True)


def _sc_aggregate(h0, h1, h2, h3, src_g, dst_g):
    out_t = tuple(jax.ShapeDtypeStruct((NA, CW), _f32) for _ in range(4))

    @functools.partial(
        pl.kernel, out_type=out_t, mesh=_sc_mesh(),
        compiler_params=_SC_PARAMS,
        scratch_types=dict(
            srcv=pltpu.VMEM((NB, BE), _i32),
            dstv=pltpu.VMEM((NB, BE), _i32),
            g0=pltpu.VMEM((BE, CW), _f32),
            g1=pltpu.VMEM((BE, CW), _f32),
            zb=pltpu.VMEM((ZR, CW), _f32),
            acc=pltpu.VMEM_SHARED((NA, CW), _f32),
            sem0=pltpu.SemaphoreType.DMA,
            sem1=pltpu.SemaphoreType.DMA,
        ),
    )
    def k(h0_hbm, h1_hbm, h2_hbm, h3_hbm, src_hbm, dst_hbm,
          o0, o1, o2, o3, srcv, dstv, g0, g1, zb, acc, sem0, sem1):
        c = lax.axis_index("c")
        s = lax.axis_index("s")
        pltpu.sync_copy(src_hbm.at[s], srcv)
        pltpu.sync_copy(dst_hbm.at[s], dstv)

        zero16 = jnp.zeros((16,), _f32)

        @pl.loop(0, ZR)
        def _(i):
            for j in range(CW // 16):
                zb[i, pl.ds(j * 16, 16)] = zero16

        hs = (h0_hbm, h1_hbm, h2_hbm, h3_hbm)
        outs = (o0, o1, o2, o3)
        for it in range(2):
            for z in range(RPT // ZR):
                pltpu.sync_copy(zb, acc.at[pl.ds(s * RPT + z * ZR, ZR)])
            plsc.subcore_barrier()
            for core in range(2):
                @pl.when(c == core)
                def _(h=hs[core * 2 + it]):
                    _edge_pass(h, srcv, dstv, g0, g1, acc, sem0, sem1)
            plsc.subcore_barrier()
            for core in range(2):
                @pl.when(c == core)
                def _(o=outs[core * 2 + it]):
                    pltpu.sync_copy(acc.at[pl.ds(s * RPT, RPT)],
                                    o.at[pl.ds(s * RPT, RPT)])
            plsc.subcore_barrier()

    return k(h0, h1, h2, h3, src_g, dst_g)


_BN = 1000


def _rsqrt_clip(d):
    return lax.rsqrt(jnp.maximum(d, 1.0))


def _tc_project(x, W, b, deg_out):
    def body(x_ref, w_ref, b_ref, d_ref, *o_refs):
        y = jnp.dot(x_ref[...], w_ref[...], preferred_element_type=_f32)
        y = (y + b_ref[...]) * _rsqrt_clip(d_ref[...])
        for ci in range(NCH):
            o_refs[ci][...] = y[:, ci * CW:(ci + 1) * CW]

    return pl.pallas_call(
        body,
        grid=(N // _BN,),
        in_specs=[
            pl.BlockSpec((_BN, D), lambda i: (i, 0)),
            pl.BlockSpec((D, D), lambda i: (0, 0)),
            pl.BlockSpec((1, D), lambda i: (0, 0)),
            pl.BlockSpec((_BN, 1), lambda i: (i, 0)),
        ],
        out_specs=[pl.BlockSpec((_BN, CW), lambda i: (i, 0))] * NCH,
        out_shape=[jax.ShapeDtypeStruct((N, CW), _f32)] * NCH,
    )(x, W, b.reshape(1, D), deg_out)


def _tc_mid(a0, a1, a2, a3, deg_in, W, b, deg_out):
    def body(a0r, a1r, a2r, a3r, di_r, w_ref, b_ref, do_r, *o_refs):
        x = jnp.concatenate([a0r[...], a1r[...], a2r[...], a3r[...]], axis=1)
        x = x * _rsqrt_clip(di_r[...])
        y = jnp.dot(x, w_ref[...], preferred_element_type=_f32) + b_ref[...]
        y = jnp.maximum(y, 0.0) * _rsqrt_clip(do_r[...])
        for ci in range(NCH):
            o_refs[ci][...] = y[:, ci * CW:(ci + 1) * CW]

    return pl.pallas_call(
        body,
        grid=(N // _BN,),
        in_specs=[pl.BlockSpec((_BN, CW), lambda i: (i, 0))] * NCH + [
            pl.BlockSpec((_BN, 1), lambda i: (i, 0)),
            pl.BlockSpec((D, D), lambda i: (0, 0)),
            pl.BlockSpec((1, D), lambda i: (0, 0)),
            pl.BlockSpec((_BN, 1), lambda i: (i, 0)),
        ],
        out_specs=[pl.BlockSpec((_BN, CW), lambda i: (i, 0))] * NCH,
        out_shape=[jax.ShapeDtypeStruct((N, CW), _f32)] * NCH,
    )(a0, a1, a2, a3, deg_in, W, b.reshape(1, D), deg_out)


def _tc_final(a0, a1, a2, a3, deg_in, W, b, half, prev=None):
    def body(a0r, a1r, a2r, a3r, di_r, w_ref, b_ref, *refs):
        o_ref = refs[-1]
        x = jnp.concatenate([a0r[...], a1r[...], a2r[...], a3r[...]], axis=1)
        x = x * _rsqrt_clip(di_r[...])
        o_ref[...] = (jnp.dot(x, w_ref[...], preferred_element_type=_f32)
                      + b_ref[...])

    nb = N // _BN
    in_specs = [pl.BlockSpec((_BN, CW), lambda i: (i, 0))] * NCH + [
        pl.BlockSpec((_BN, 1), lambda i: (i, 0)),
        pl.BlockSpec((D, D), lambda i: (0, 0)),
        pl.BlockSpec((1, D), lambda i: (0, 0)),
    ]
    args = [a0, a1, a2, a3, deg_in, W, b.reshape(1, D)]
    aliases = {}
    if prev is not None:
        in_specs = in_specs + [pl.BlockSpec(memory_space=pl.ANY)]
        args.append(prev)
        aliases = {len(args) - 1: 0}
    return pl.pallas_call(
        body,
        grid=(nb,),
        in_specs=in_specs,
        out_specs=pl.BlockSpec((_BN, D), lambda i: (i + half * nb, 0)),
        out_shape=jax.ShapeDtypeStruct((2 * N, D), _f32),
        input_output_aliases=aliases,
    )(*args)


def kernel(x_user, x_item, edge_rates, edge_rated_by,
           W_in_user, b_in_user, W_in_item, b_in_item,
           W1_rates, b1_rates, W1_rated_by, b1_rated_by,
           W2_rates, b2_rates, W2_rated_by, b2_rated_by):
    er_src = edge_rates[0].astype(_i32)
    er_dst = edge_rates[1].astype(_i32)
    eb_src = edge_rated_by[0].astype(_i32)
    eb_dst = edge_rated_by[1].astype(_i32)

    hu_out, hi_in, hi_out, hu_in = _sc_degrees(
        er_src.reshape(HT, HEPT), er_dst.reshape(HT, HEPT),
        eb_src.reshape(HT, HEPT), eb_dst.reshape(HT, HEPT))
    du_out = hu_out.reshape(NH, 1)
    di_in = hi_in.reshape(NH, 1)
    di_out = hi_out.reshape(NH, 1)
    du_in = hu_in.reshape(NH, 1)

    srg = er_src.reshape(NS, NB, BE)
    drg = er_dst.reshape(NS, NB, BE)
    srb = eb_src.reshape(NS, NB, BE)
    drb = eb_dst.reshape(NS, NB, BE)

    hu = _tc_project(x_user, W_in_user, b_in_user, du_out)
    hi = _tc_project(x_item, W_in_item, b_in_item, di_out)

    ai1 = _sc_aggregate(*hu, srg, drg)
    au1 = _sc_aggregate(*hi, srb, drb)

    h1i = _tc_mid(*ai1, di_in, W1_rates, b1_rates, di_out)
    h1u = _tc_mid(*au1, du_in, W1_rated_by, b1_rated_by, du_out)

    ai2 = _sc_aggregate(*h1u, srg, drg)
    au2 = _sc_aggregate(*h1i, srb, drb)

    out = _tc_final(*au2, du_in, W2_rated_by, b2_rated_by, half=0)
    out = _tc_final(*ai2, di_in, W2_rates, b2_rates, half=1, prev=out)
    return out


# speedup vs baseline: 1.1868x; 1.0007x over previous
import functools

import jax
import jax.numpy as jnp
from jax import lax
from jax.experimental import pallas as pl
from jax.experimental.pallas import tpu as pltpu
from jax.experimental.pallas import tpu_sc as plsc

N = 10000
E = 80000
D = 512
NCH = 4
CW = D // NCH
NS = 16
EPT = E // NS
BE = 50
NB = EPT // BE
NA = 10240
RPT = NA // NS
ZR = 32

HT = 8
HEPT = E // HT
NH = 10240
HS = NH // NS

_f32 = jnp.float32
_bf16 = jnp.bfloat16
_i32 = jnp.int32


def _sc_mesh():
    return plsc.VectorSubcoreMesh(core_axis_name="c", subcore_axis_name="s",
                                  num_cores=2, num_subcores=NS)


_SC_PARAMS = pltpu.CompilerParams(needs_layout_passes=False)


def _sc_degrees(ia, ib, ic, id_):
    out_t = tuple(jax.ShapeDtypeStruct((NH,), _f32) for _ in range(4))

    @functools.partial(
        pl.kernel, out_type=out_t, mesh=_sc_mesh(),
        compiler_params=_SC_PARAMS,
        scratch_types=dict(
            idxv=pltpu.VMEM((HEPT,), _i32),
            hist=pltpu.VMEM((NH,), _f32),
            red=pltpu.VMEM((HT, HS), _f32),
            res=pltpu.VMEM((HS,), _f32),
            shist=pltpu.VMEM_SHARED((2, HT, NH), _f32),
        ),
    )
    def k(a_hbm, b_hbm, c_hbm, d_hbm, oa, ob, oc, od,
          idxv, hist, red, res, shist):
        c = lax.axis_index("c")
        s = lax.axis_index("s")
        grp = s // HT
        row = s - grp * HT

        for cc, g, ref in ((0, 0, a_hbm), (0, 1, b_hbm),
                           (1, 0, c_hbm), (1, 1, d_hbm)):
            @pl.when(jnp.logical_and(c == cc, grp == g))
            def _(ref=ref):
                pltpu.sync_copy(ref.at[row], idxv)

        zero16 = jnp.zeros((16,), _f32)

        @pl.loop(0, NH, step=16)
        def _(i):
            hist[pl.ds(i, 16)] = zero16

        ones16 = jnp.ones((16,), _f32)

        @pl.loop(0, HEPT, step=16)
        def _(i):
            idx16 = idxv[pl.ds(i, 16)]
            plsc.addupdate_scatter(hist, [idx16], ones16)

        pltpu.sync_copy(hist, shist.at[grp, row])
        plsc.subcore_barrier()

        for g in range(2):
            pltpu.sync_copy(shist.at[g, :, pl.ds(s * HS, HS)], red)

            @pl.loop(0, HS, step=16)
            def _(i):
                acc = red[0, pl.ds(i, 16)]
                for r in range(1, HT):
                    acc = acc + red[r, pl.ds(i, 16)]
                res[pl.ds(i, 16)] = acc

            for cc, out in ((0, (oa, ob)[g]), (1, (oc, od)[g])):
                @pl.when(c == cc)
                def _(out=out):
                    pltpu.sync_copy(res, out.at[pl.ds(s * HS, HS)])

    return k(ia, ib, ic, id_)


def _edge_pass(h, srcv, dstv, g0, g1, acc, sem0, sem1):
    pltpu.async_copy(h.at[srcv.at[0]], g0, sem0)

    @pl.loop(0, NB, step=2)
    def _(j):
        pltpu.async_copy(h.at[srcv.at[j + 1]], g1, sem1)
        pltpu.make_async_copy(h.at[srcv.at[j]], g0, sem0).wait()
        pltpu.sync_copy(g0, acc.at[dstv.at[j]], add=True)

        @pl.when(j + 2 < NB)
        def _():
            pltpu.async_copy(h.at[srcv.at[j + 2]], g0, sem0)

        pltpu.make_async_copy(h.at[srcv.at[j + 1]], g1, sem1).wait()
        pltpu.sync_copy(g1, acc.at[dstv.at[j + 1]], add=True)


def _sc_aggregate(h0, h1, h2, h3, src_g, dst_g):
    out_t = tuple(jax.ShapeDtypeStruct((NA, CW), _f32) for _ in range(4))

    @functools.partial(
        pl.kernel, out_type=out_t, mesh=_sc_mesh(),
        compiler_params=_SC_PARAMS,
        scratch_types=dict(
            srcv=pltpu.VMEM((NB, BE), _i32),
            dstv=pltpu.VMEM((NB, BE), _i32),
            g0=pltpu.VMEM((BE, CW), _f32),
            g1=pltpu.VMEM((BE, CW), _f32),
            zb=pltpu.VMEM((ZR, CW), _f32),
            acc=pltpu.VMEM_SHARED((NA, CW), _f32),
            sem0=pltpu.SemaphoreType.DMA,
            sem1=pltpu.SemaphoreType.DMA,
        ),
    )
    def k(h0_hbm, h1_hbm, h2_hbm, h3_hbm, src_hbm, dst_hbm,
          o0, o1, o2, o3, srcv, dstv, g0, g1, zb, acc, sem0, sem1):
        c = lax.axis_index("c")
        s = lax.axis_index("s")
        pltpu.sync_copy(src_hbm.at[s], srcv)
        pltpu.sync_copy(dst_hbm.at[s], dstv)

        zero16 = jnp.zeros((16,), _f32)

        @pl.loop(0, ZR)
        def _(i):
            for j in range(CW // 16):
                zb[i, pl.ds(j * 16, 16)] = zero16

        hs = (h0_hbm, h1_hbm, h2_hbm, h3_hbm)
        outs = (o0, o1, o2, o3)
        for it in range(2):
            for z in range(RPT // ZR):
                pltpu.sync_copy(zb, acc.at[pl.ds(s * RPT + z * ZR, ZR)])
            plsc.subcore_barrier()
            for core in range(2):
                @pl.when(c == core)
                def _(h=hs[core * 2 + it]):
                    _edge_pass(h, srcv, dstv, g0, g1, acc, sem0, sem1)
            plsc.subcore_barrier()
            for core in range(2):
                @pl.when(c == core)
                def _(o=outs[core * 2 + it]):
                    pltpu.sync_copy(acc.at[pl.ds(s * RPT, RPT)],
                                    o.at[pl.ds(s * RPT, RPT)])
            plsc.subcore_barrier()

    return k(h0, h1, h2, h3, src_g, dst_g)


_BN = 1000


def _rsqrt_clip(d):
    return lax.rsqrt(jnp.maximum(d, 1.0))


def _tc_project(x, W, b, deg_out):
    def body(x_ref, w_ref, b_ref, d_ref, *o_refs):
        y = jnp.dot(x_ref[...].astype(_bf16), w_ref[...].astype(_bf16),
                    preferred_element_type=_f32)
        y = (y + b_ref[...]) * _rsqrt_clip(d_ref[...])
        for ci in range(NCH):
            o_refs[ci][...] = y[:, ci * CW:(ci + 1) * CW]

    return pl.pallas_call(
        body,
        grid=(N // _BN,),
        in_specs=[
            pl.BlockSpec((_BN, D), lambda i: (i, 0)),
            pl.BlockSpec((D, D), lambda i: (0, 0)),
            pl.BlockSpec((1, D), lambda i: (0, 0)),
            pl.BlockSpec((_BN, 1), lambda i: (i, 0)),
        ],
        out_specs=[pl.BlockSpec((_BN, CW), lambda i: (i, 0))] * NCH,
        out_shape=[jax.ShapeDtypeStruct((N, CW), _f32)] * NCH,
    )(x, W, b.reshape(1, D), deg_out)


def _tc_mid(a0, a1, a2, a3, deg_in, W, b, deg_out):
    def body(a0r, a1r, a2r, a3r, di_r, w_ref, b_ref, do_r, *o_refs):
        x = jnp.concatenate([a0r[...], a1r[...], a2r[...], a3r[...]], axis=1)
        x = (x * _rsqrt_clip(di_r[...])).astype(_bf16)
        y = (jnp.dot(x, w_ref[...].astype(_bf16), preferred_element_type=_f32)
             + b_ref[...])
        y = jnp.maximum(y, 0.0) * _rsqrt_clip(do_r[...])
        for ci in range(NCH):
            o_refs[ci][...] = y[:, ci * CW:(ci + 1) * CW]

    return pl.pallas_call(
        body,
        grid=(N // _BN,),
        in_specs=[pl.BlockSpec((_BN, CW), lambda i: (i, 0))] * NCH + [
            pl.BlockSpec((_BN, 1), lambda i: (i, 0)),
            pl.BlockSpec((D, D), lambda i: (0, 0)),
            pl.BlockSpec((1, D), lambda i: (0, 0)),
            pl.BlockSpec((_BN, 1), lambda i: (i, 0)),
        ],
        out_specs=[pl.BlockSpec((_BN, CW), lambda i: (i, 0))] * NCH,
        out_shape=[jax.ShapeDtypeStruct((N, CW), _f32)] * NCH,
    )(a0, a1, a2, a3, deg_in, W, b.reshape(1, D), deg_out)


def _tc_final(a0, a1, a2, a3, deg_in, W, b, half, prev=None):
    def body(a0r, a1r, a2r, a3r, di_r, w_ref, b_ref, *refs):
        o_ref = refs[-1]
        x = jnp.concatenate([a0r[...], a1r[...], a2r[...], a3r[...]], axis=1)
        x = (x * _rsqrt_clip(di_r[...])).astype(_bf16)
        o_ref[...] = (jnp.dot(x, w_ref[...].astype(_bf16),
                              preferred_element_type=_f32) + b_ref[...])

    nb = N // _BN
    in_specs = [pl.BlockSpec((_BN, CW), lambda i: (i, 0))] * NCH + [
        pl.BlockSpec((_BN, 1), lambda i: (i, 0)),
        pl.BlockSpec((D, D), lambda i: (0, 0)),
        pl.BlockSpec((1, D), lambda i: (0, 0)),
    ]
    args = [a0, a1, a2, a3, deg_in, W, b.reshape(1, D)]
    aliases = {}
    if prev is not None:
        in_specs = in_specs + [pl.BlockSpec(memory_space=pl.ANY)]
        args.append(prev)
        aliases = {len(args) - 1: 0}
    return pl.pallas_call(
        body,
        grid=(nb,),
        in_specs=in_specs,
        out_specs=pl.BlockSpec((_BN, D), lambda i: (i + half * nb, 0)),
        out_shape=jax.ShapeDtypeStruct((2 * N, D), _f32),
        input_output_aliases=aliases,
    )(*args)


def kernel(x_user, x_item, edge_rates, edge_rated_by,
           W_in_user, b_in_user, W_in_item, b_in_item,
           W1_rates, b1_rates, W1_rated_by, b1_rated_by,
           W2_rates, b2_rates, W2_rated_by, b2_rated_by):
    er_src = edge_rates[0].astype(_i32)
    er_dst = edge_rates[1].astype(_i32)
    eb_src = edge_rated_by[0].astype(_i32)
    eb_dst = edge_rated_by[1].astype(_i32)

    hu_out, hi_in, hi_out, hu_in = _sc_degrees(
        er_src.reshape(HT, HEPT), er_dst.reshape(HT, HEPT),
        eb_src.reshape(HT, HEPT), eb_dst.reshape(HT, HEPT))
    du_out = hu_out.reshape(NH, 1)
    di_in = hi_in.reshape(NH, 1)
    di_out = hi_out.reshape(NH, 1)
    du_in = hu_in.reshape(NH, 1)

    srg = er_src.reshape(NS, NB, BE)
    drg = er_dst.reshape(NS, NB, BE)
    srb = eb_src.reshape(NS, NB, BE)
    drb = eb_dst.reshape(NS, NB, BE)

    hu = _tc_project(x_user, W_in_user, b_in_user, du_out)
    hi = _tc_project(x_item, W_in_item, b_in_item, di_out)

    ai1 = _sc_aggregate(*hu, srg, drg)
    au1 = _sc_aggregate(*hi, srb, drb)

    h1i = _tc_mid(*ai1, di_in, W1_rates, b1_rates, di_out)
    h1u = _tc_mid(*au1, du_in, W1_rated_by, b1_rated_by, du_out)

    ai2 = _sc_aggregate(*h1u, srg, drg)
    au2 = _sc_aggregate(*h1i, srb, drb)

    out = _tc_final(*au2, du_in, W2_rated_by, b2_rated_by, half=0)
    out = _tc_final(*ai2, di_in, W2_rates, b2_rates, half=1, prev=out)
    return out


# speedup vs baseline: 1.3803x; 1.1631x over previous
import functools

import jax
import jax.numpy as jnp
from jax import lax
from jax.experimental import pallas as pl
from jax.experimental.pallas import tpu as pltpu
from jax.experimental.pallas import tpu_sc as plsc

N = 10000
E = 80000
D = 512
NCH = 4
CW = D // NCH
NS = 16
EPT = E // NS
BE = 100
NB = EPT // BE
NA = 10240
RPT = NA // NS
ZR = 8

HT = 8
HEPT = E // HT
NH = 10240
HS = NH // NS

_f32 = jnp.float32
_bf16 = jnp.bfloat16
_i32 = jnp.int32


def _sc_mesh():
    return plsc.VectorSubcoreMesh(core_axis_name="c", subcore_axis_name="s",
                                  num_cores=2, num_subcores=NS)


_SC_PARAMS = pltpu.CompilerParams(needs_layout_passes=False)


def _sc_degrees(ia, ib, ic, id_):
    out_t = tuple(jax.ShapeDtypeStruct((NH,), _f32) for _ in range(4))

    @functools.partial(
        pl.kernel, out_type=out_t, mesh=_sc_mesh(),
        compiler_params=_SC_PARAMS,
        scratch_types=dict(
            idxv=pltpu.VMEM((HEPT,), _i32),
            hist=pltpu.VMEM((NH,), _f32),
            red=pltpu.VMEM((HT, HS), _f32),
            res=pltpu.VMEM((HS,), _f32),
            shist=pltpu.VMEM_SHARED((2, HT, NH), _f32),
        ),
    )
    def k(a_hbm, b_hbm, c_hbm, d_hbm, oa, ob, oc, od,
          idxv, hist, red, res, shist):
        c = lax.axis_index("c")
        s = lax.axis_index("s")
        grp = s // HT
        row = s - grp * HT

        for cc, g, ref in ((0, 0, a_hbm), (0, 1, b_hbm),
                           (1, 0, c_hbm), (1, 1, d_hbm)):
            @pl.when(jnp.logical_and(c == cc, grp == g))
            def _(ref=ref):
                pltpu.sync_copy(ref.at[row], idxv)

        zero16 = jnp.zeros((16,), _f32)

        @pl.loop(0, NH, step=16)
        def _(i):
            hist[pl.ds(i, 16)] = zero16

        ones16 = jnp.ones((16,), _f32)

        @pl.loop(0, HEPT, step=16)
        def _(i):
            idx16 = idxv[pl.ds(i, 16)]
            plsc.addupdate_scatter(hist, [idx16], ones16)

        pltpu.sync_copy(hist, shist.at[grp, row])
        plsc.subcore_barrier()

        for g in range(2):
            pltpu.sync_copy(shist.at[g, :, pl.ds(s * HS, HS)], red)

            @pl.loop(0, HS, step=16)
            def _(i):
                acc = red[0, pl.ds(i, 16)]
                for r in range(1, HT):
                    acc = acc + red[r, pl.ds(i, 16)]
                res[pl.ds(i, 16)] = acc

            for cc, out in ((0, (oa, ob)[g]), (1, (oc, od)[g])):
                @pl.when(c == cc)
                def _(out=out):
                    pltpu.sync_copy(res, out.at[pl.ds(s * HS, HS)])

    return k(ia, ib, ic, id_)


def _edge_pass(h, srcv, dstv, g0, g1, acc, sem0, sem1):
    pltpu.async_copy(h.at[srcv.at[0]], g0, sem0)

    @pl.loop(0, NB, step=2)
    def _(j):
        pltpu.async_copy(h.at[srcv.at[j + 1]], g1, sem1)
        pltpu.make_async_copy(h.at[srcv.at[j]], g0, sem0).wait()
        pltpu.sync_copy(g0, acc.at[dstv.at[j]], add=True)

        @pl.when(j + 2 < NB)
        def _():
            pltpu.async_copy(h.at[srcv.at[j + 2]], g0, sem0)

        pltpu.make_async_copy(h.at[srcv.at[j + 1]], g1, sem1).wait()
        pltpu.sync_copy(g1, acc.at[dstv.at[j + 1]], add=True)


def _sc_aggregate(h0, h1, h2, h3, src_g, dst_g):
    out_t = tuple(jax.ShapeDtypeStruct((NA, CW), _f32) for _ in range(4))

    @functools.partial(
        pl.kernel, out_type=out_t, mesh=_sc_mesh(),
        compiler_params=_SC_PARAMS,
        scratch_types=dict(
            srcv=pltpu.VMEM((NB, BE), _i32),
            dstv=pltpu.VMEM((NB, BE), _i32),
            g0=pltpu.VMEM((BE, CW), _f32),
            g1=pltpu.VMEM((BE, CW), _f32),
            zb=pltpu.VMEM((ZR, CW), _f32),
            acc=pltpu.VMEM_SHARED((NA, CW), _f32),
            sem0=pltpu.SemaphoreType.DMA,
            sem1=pltpu.SemaphoreType.DMA,
        ),
    )
    def k(h0_hbm, h1_hbm, h2_hbm, h3_hbm, src_hbm, dst_hbm,
          o0, o1, o2, o3, srcv, dstv, g0, g1, zb, acc, sem0, sem1):
        c = lax.axis_index("c")
        s = lax.axis_index("s")
        pltpu.sync_copy(src_hbm.at[s], srcv)
        pltpu.sync_copy(dst_hbm.at[s], dstv)

        zero16 = jnp.zeros((16,), _f32)

        @pl.loop(0, ZR)
        def _(i):
            for j in range(CW // 16):
                zb[i, pl.ds(j * 16, 16)] = zero16

        hs = (h0_hbm, h1_hbm, h2_hbm, h3_hbm)
        outs = (o0, o1, o2, o3)
        for it in range(2):
            for z in range(RPT // ZR):
                pltpu.sync_copy(zb, acc.at[pl.ds(s * RPT + z * ZR, ZR)])
            plsc.subcore_barrier()
            for core in range(2):
                @pl.when(c == core)
                def _(h=hs[core * 2 + it]):
                    _edge_pass(h, srcv, dstv, g0, g1, acc, sem0, sem1)
            plsc.subcore_barrier()
            for core in range(2):
                @pl.when(c == core)
                def _(o=outs[core * 2 + it]):
                    pltpu.sync_copy(acc.at[pl.ds(s * RPT, RPT)],
                                    o.at[pl.ds(s * RPT, RPT)])
            plsc.subcore_barrier()

    return k(h0, h1, h2, h3, src_g, dst_g)


_BN = 1000


def _rsqrt_clip(d):
    return lax.rsqrt(jnp.maximum(d, 1.0))


def _tc_project(x, W, b, deg_out):
    def body(x_ref, w_ref, b_ref, d_ref, *o_refs):
        y = jnp.dot(x_ref[...].astype(_bf16), w_ref[...].astype(_bf16),
                    preferred_element_type=_f32)
        y = (y + b_ref[...]) * _rsqrt_clip(d_ref[...])
        for ci in range(NCH):
            o_refs[ci][...] = y[:, ci * CW:(ci + 1) * CW]

    return pl.pallas_call(
        body,
        grid=(N // _BN,),
        in_specs=[
            pl.BlockSpec((_BN, D), lambda i: (i, 0)),
            pl.BlockSpec((D, D), lambda i: (0, 0)),
            pl.BlockSpec((1, D), lambda i: (0, 0)),
            pl.BlockSpec((_BN, 1), lambda i: (i, 0)),
        ],
        out_specs=[pl.BlockSpec((_BN, CW), lambda i: (i, 0))] * NCH,
        out_shape=[jax.ShapeDtypeStruct((N, CW), _f32)] * NCH,
    )(x, W, b.reshape(1, D), deg_out)


def _tc_mid(a0, a1, a2, a3, deg_in, W, b, deg_out):
    def body(a0r, a1r, a2r, a3r, di_r, w_ref, b_ref, do_r, *o_refs):
        x = jnp.concatenate([a0r[...], a1r[...], a2r[...], a3r[...]], axis=1)
        x = (x * _rsqrt_clip(di_r[...])).astype(_bf16)
        y = (jnp.dot(x, w_ref[...].astype(_bf16), preferred_element_type=_f32)
             + b_ref[...])
        y = jnp.maximum(y, 0.0) * _rsqrt_clip(do_r[...])
        for ci in range(NCH):
            o_refs[ci][...] = y[:, ci * CW:(ci + 1) * CW]

    return pl.pallas_call(
        body,
        grid=(N // _BN,),
        in_specs=[pl.BlockSpec((_BN, CW), lambda i: (i, 0))] * NCH + [
            pl.BlockSpec((_BN, 1), lambda i: (i, 0)),
            pl.BlockSpec((D, D), lambda i: (0, 0)),
            pl.BlockSpec((1, D), lambda i: (0, 0)),
            pl.BlockSpec((_BN, 1), lambda i: (i, 0)),
        ],
        out_specs=[pl.BlockSpec((_BN, CW), lambda i: (i, 0))] * NCH,
        out_shape=[jax.ShapeDtypeStruct((N, CW), _f32)] * NCH,
    )(a0, a1, a2, a3, deg_in, W, b.reshape(1, D), deg_out)


def _tc_final(a0, a1, a2, a3, deg_in, W, b, half, prev=None):
    def body(a0r, a1r, a2r, a3r, di_r, w_ref, b_ref, *refs):
        o_ref = refs[-1]
        x = jnp.concatenate([a0r[...], a1r[...], a2r[...], a3r[...]], axis=1)
        x = (x * _rsqrt_clip(di_r[...])).astype(_bf16)
        o_ref[...] = (jnp.dot(x, w_ref[...].astype(_bf16),
                              preferred_element_type=_f32) + b_ref[...])

    nb = N // _BN
    in_specs = [pl.BlockSpec((_BN, CW), lambda i: (i, 0))] * NCH + [
        pl.BlockSpec((_BN, 1), lambda i: (i, 0)),
        pl.BlockSpec((D, D), lambda i: (0, 0)),
        pl.BlockSpec((1, D), lambda i: (0, 0)),
    ]
    args = [a0, a1, a2, a3, deg_in, W, b.reshape(1, D)]
    aliases = {}
    if prev is not None:
        in_specs = in_specs + [pl.BlockSpec(memory_space=pl.ANY)]
        args.append(prev)
        aliases = {len(args) - 1: 0}
    return pl.pallas_call(
        body,
        grid=(nb,),
        in_specs=in_specs,
        out_specs=pl.BlockSpec((_BN, D), lambda i: (i + half * nb, 0)),
        out_shape=jax.ShapeDtypeStruct((2 * N, D), _f32),
        input_output_aliases=aliases,
    )(*args)


def kernel(x_user, x_item, edge_rates, edge_rated_by,
           W_in_user, b_in_user, W_in_item, b_in_item,
           W1_rates, b1_rates, W1_rated_by, b1_rated_by,
           W2_rates, b2_rates, W2_rated_by, b2_rated_by):
    er_src = edge_rates[0].astype(_i32)
    er_dst = edge_rates[1].astype(_i32)
    eb_src = edge_rated_by[0].astype(_i32)
    eb_dst = edge_rated_by[1].astype(_i32)

    hu_out, hi_in, hi_out, hu_in = _sc_degrees(
        er_src.reshape(HT, HEPT), er_dst.reshape(HT, HEPT),
        eb_src.reshape(HT, HEPT), eb_dst.reshape(HT, HEPT))
    du_out = hu_out.reshape(NH, 1)
    di_in = hi_in.reshape(NH, 1)
    di_out = hi_out.reshape(NH, 1)
    du_in = hu_in.reshape(NH, 1)

    srg = er_src.reshape(NS, NB, BE)
    drg = er_dst.reshape(NS, NB, BE)
    srb = eb_src.reshape(NS, NB, BE)
    drb = eb_dst.reshape(NS, NB, BE)

    hu = _tc_project(x_user, W_in_user, b_in_user, du_out)
    hi = _tc_project(x_item, W_in_item, b_in_item, di_out)

    ai1 = _sc_aggregate(*hu, srg, drg)
    au1 = _sc_aggregate(*hi, srb, drb)

    h1i = _tc_mid(*ai1, di_in, W1_rates, b1_rates, di_out)
    h1u = _tc_mid(*au1, du_in, W1_rated_by, b1_rated_by, du_out)

    ai2 = _sc_aggregate(*h1u, srg, drg)
    au2 = _sc_aggregate(*h1i, srb, drb)

    out = _tc_final(*au2, du_in, W2_rated_by, b2_rated_by, half=0)
    out = _tc_final(*ai2, di_in, W2_rates, b2_rates, half=1, prev=out)
    return out


# speedup vs baseline: 1.4412x; 1.0441x over previous
import functools

import jax
import jax.numpy as jnp
from jax import lax
from jax.experimental import pallas as pl
from jax.experimental.pallas import tpu as pltpu
from jax.experimental.pallas import tpu_sc as plsc

N = 10000
E = 80000
D = 512
NCH = 4
CW = D // NCH
NS = 16
EPT = E // NS
BE = 100
NB = EPT // BE
NA = 10240
RPT = NA // NS
ZR = 64

HT = 8
HEPT = E // HT
NH = 10240
HS = NH // NS

_f32 = jnp.float32
_bf16 = jnp.bfloat16
_i32 = jnp.int32


def _sc_mesh():
    return plsc.VectorSubcoreMesh(core_axis_name="c", subcore_axis_name="s",
                                  num_cores=2, num_subcores=NS)


_SC_PARAMS = pltpu.CompilerParams(needs_layout_passes=False)


def _sc_degrees(ia, ib, ic, id_):
    out_t = tuple(jax.ShapeDtypeStruct((NH,), _f32) for _ in range(4))

    @functools.partial(
        pl.kernel, out_type=out_t, mesh=_sc_mesh(),
        compiler_params=_SC_PARAMS,
        scratch_types=dict(
            idxv=pltpu.VMEM((HEPT,), _i32),
            hist=pltpu.VMEM((NH,), _f32),
            red=pltpu.VMEM((HT, HS), _f32),
            res=pltpu.VMEM((HS,), _f32),
            shist=pltpu.VMEM_SHARED((2, HT, NH), _f32),
        ),
    )
    def k(a_hbm, b_hbm, c_hbm, d_hbm, oa, ob, oc, od,
          idxv, hist, red, res, shist):
        c = lax.axis_index("c")
        s = lax.axis_index("s")
        grp = s // HT
        row = s - grp * HT

        for cc, g, ref in ((0, 0, a_hbm), (0, 1, b_hbm),
                           (1, 0, c_hbm), (1, 1, d_hbm)):
            @pl.when(jnp.logical_and(c == cc, grp == g))
            def _(ref=ref):
                pltpu.sync_copy(ref.at[row], idxv)

        zero16 = jnp.zeros((16,), _f32)

        @pl.loop(0, NH, step=16)
        def _(i):
            hist[pl.ds(i, 16)] = zero16

        ones16 = jnp.ones((16,), _f32)

        @pl.loop(0, HEPT, step=16)
        def _(i):
            idx16 = idxv[pl.ds(i, 16)]
            plsc.addupdate_scatter(hist, [idx16], ones16)

        pltpu.sync_copy(hist, shist.at[grp, row])
        plsc.subcore_barrier()

        for g in range(2):
            pltpu.sync_copy(shist.at[g, :, pl.ds(s * HS, HS)], red)

            @pl.loop(0, HS, step=16)
            def _(i):
                acc = red[0, pl.ds(i, 16)]
                for r in range(1, HT):
                    acc = acc + red[r, pl.ds(i, 16)]
                res[pl.ds(i, 16)] = acc

            for cc, out in ((0, (oa, ob)[g]), (1, (oc, od)[g])):
                @pl.when(c == cc)
                def _(out=out):
                    pltpu.sync_copy(res, out.at[pl.ds(s * HS, HS)])

    return k(ia, ib, ic, id_)


def _edge_pass(h, srcv, dstv, g0, g1, acc, sem0, sem1):
    pltpu.async_copy(h.at[srcv.at[0]], g0, sem0)

    @pl.loop(0, NB, step=2)
    def _(j):
        pltpu.async_copy(h.at[srcv.at[j + 1]], g1, sem1)
        pltpu.make_async_copy(h.at[srcv.at[j]], g0, sem0).wait()
        pltpu.sync_copy(g0, acc.at[dstv.at[j]], add=True)

        @pl.when(j + 2 < NB)
        def _():
            pltpu.async_copy(h.at[srcv.at[j + 2]], g0, sem0)

        pltpu.make_async_copy(h.at[srcv.at[j + 1]], g1, sem1).wait()
        pltpu.sync_copy(g1, acc.at[dstv.at[j + 1]], add=True)


def _sc_aggregate(h0, h1, h2, h3, src_g, dst_g):
    out_t = tuple(jax.ShapeDtypeStruct((NA, CW), _f32) for _ in range(4))

    @functools.partial(
        pl.kernel, out_type=out_t, mesh=_sc_mesh(),
        compiler_params=_SC_PARAMS,
        scratch_types=dict(
            srcv=pltpu.VMEM((NB, BE), _i32),
            dstv=pltpu.VMEM((NB, BE), _i32),
            g0=pltpu.VMEM((BE, CW), _f32),
            g1=pltpu.VMEM((BE, CW), _f32),
            acc=pltpu.VMEM_SHARED((NA, CW), _f32),
            sem0=pltpu.SemaphoreType.DMA,
            sem1=pltpu.SemaphoreType.DMA,
        ),
    )
    def k(h0_hbm, h1_hbm, h2_hbm, h3_hbm, src_hbm, dst_hbm,
          o0, o1, o2, o3, srcv, dstv, g0, g1, acc, sem0, sem1):
        c = lax.axis_index("c")
        s = lax.axis_index("s")
        pltpu.sync_copy(src_hbm.at[s], srcv)
        pltpu.sync_copy(dst_hbm.at[s], dstv)

        zero16 = jnp.zeros((16,), _f32)

        hs = (h0_hbm, h1_hbm, h2_hbm, h3_hbm)
        outs = (o0, o1, o2, o3)
        for it in range(2):
            @pl.loop(0, ZR)
            def _(i):
                for j in range(CW // 16):
                    g0[i, pl.ds(j * 16, 16)] = zero16

            for z in range(RPT // ZR):
                pltpu.sync_copy(g0.at[pl.ds(0, ZR)],
                                acc.at[pl.ds(s * RPT + z * ZR, ZR)])
            plsc.subcore_barrier()
            for core in range(2):
                @pl.when(c == core)
                def _(h=hs[core * 2 + it]):
                    _edge_pass(h, srcv, dstv, g0, g1, acc, sem0, sem1)
            plsc.subcore_barrier()
            for core in range(2):
                @pl.when(c == core)
                def _(o=outs[core * 2 + it]):
                    pltpu.sync_copy(acc.at[pl.ds(s * RPT, RPT)],
                                    o.at[pl.ds(s * RPT, RPT)])
            plsc.subcore_barrier()

    return k(h0, h1, h2, h3, src_g, dst_g)


_BN = 1000


def _rsqrt_clip(d):
    return lax.rsqrt(jnp.maximum(d, 1.0))


def _tc_project(x, W, b, deg_out):
    def body(x_ref, w_ref, b_ref, d_ref, *o_refs):
        y = jnp.dot(x_ref[...].astype(_bf16), w_ref[...].astype(_bf16),
                    preferred_element_type=_f32)
        y = (y + b_ref[...]) * _rsqrt_clip(d_ref[...])
        for ci in range(NCH):
            o_refs[ci][...] = y[:, ci * CW:(ci + 1) * CW]

    return pl.pallas_call(
        body,
        grid=(N // _BN,),
        in_specs=[
            pl.BlockSpec((_BN, D), lambda i: (i, 0)),
            pl.BlockSpec((D, D), lambda i: (0, 0)),
            pl.BlockSpec((1, D), lambda i: (0, 0)),
            pl.BlockSpec((_BN, 1), lambda i: (i, 0)),
        ],
        out_specs=[pl.BlockSpec((_BN, CW), lambda i: (i, 0))] * NCH,
        out_shape=[jax.ShapeDtypeStruct((N, CW), _f32)] * NCH,
    )(x, W, b.reshape(1, D), deg_out)


def _tc_mid(a0, a1, a2, a3, deg_in, W, b, deg_out):
    def body(a0r, a1r, a2r, a3r, di_r, w_ref, b_ref, do_r, *o_refs):
        x = jnp.concatenate([a0r[...], a1r[...], a2r[...], a3r[...]], axis=1)
        x = (x * _rsqrt_clip(di_r[...])).astype(_bf16)
        y = (jnp.dot(x, w_ref[...].astype(_bf16), preferred_element_type=_f32)
             + b_ref[...])
        y = jnp.maximum(y, 0.0) * _rsqrt_clip(do_r[...])
        for ci in range(NCH):
            o_refs[ci][...] = y[:, ci * CW:(ci + 1) * CW]

    return pl.pallas_call(
        body,
        grid=(N // _BN,),
        in_specs=[pl.BlockSpec((_BN, CW), lambda i: (i, 0))] * NCH + [
            pl.BlockSpec((_BN, 1), lambda i: (i, 0)),
            pl.BlockSpec((D, D), lambda i: (0, 0)),
            pl.BlockSpec((1, D), lambda i: (0, 0)),
            pl.BlockSpec((_BN, 1), lambda i: (i, 0)),
        ],
        out_specs=[pl.BlockSpec((_BN, CW), lambda i: (i, 0))] * NCH,
        out_shape=[jax.ShapeDtypeStruct((N, CW), _f32)] * NCH,
    )(a0, a1, a2, a3, deg_in, W, b.reshape(1, D), deg_out)


def _tc_final(a0, a1, a2, a3, deg_in, W, b, half, prev=None):
    def body(a0r, a1r, a2r, a3r, di_r, w_ref, b_ref, *refs):
        o_ref = refs[-1]
        x = jnp.concatenate([a0r[...], a1r[...], a2r[...], a3r[...]], axis=1)
        x = (x * _rsqrt_clip(di_r[...])).astype(_bf16)
        o_ref[...] = (jnp.dot(x, w_ref[...].astype(_bf16),
                              preferred_element_type=_f32) + b_ref[...])

    nb = N // _BN
    in_specs = [pl.BlockSpec((_BN, CW), lambda i: (i, 0))] * NCH + [
        pl.BlockSpec((_BN, 1), lambda i: (i, 0)),
        pl.BlockSpec((D, D), lambda i: (0, 0)),
        pl.BlockSpec((1, D), lambda i: (0, 0)),
    ]
    args = [a0, a1, a2, a3, deg_in, W, b.reshape(1, D)]
    aliases = {}
    if prev is not None:
        in_specs = in_specs + [pl.BlockSpec(memory_space=pl.ANY)]
        args.append(prev)
        aliases = {len(args) - 1: 0}
    return pl.pallas_call(
        body,
        grid=(nb,),
        in_specs=in_specs,
        out_specs=pl.BlockSpec((_BN, D), lambda i: (i + half * nb, 0)),
        out_shape=jax.ShapeDtypeStruct((2 * N, D), _f32),
        input_output_aliases=aliases,
    )(*args)


def kernel(x_user, x_item, edge_rates, edge_rated_by,
           W_in_user, b_in_user, W_in_item, b_in_item,
           W1_rates, b1_rates, W1_rated_by, b1_rated_by,
           W2_rates, b2_rates, W2_rated_by, b2_rated_by):
    er_src = edge_rates[0].astype(_i32)
    er_dst = edge_rates[1].astype(_i32)
    eb_src = edge_rated_by[0].astype(_i32)
    eb_dst = edge_rated_by[1].astype(_i32)

    hu_out, hi_in, hi_out, hu_in = _sc_degrees(
        er_src.reshape(HT, HEPT), er_dst.reshape(HT, HEPT),
        eb_src.reshape(HT, HEPT), eb_dst.reshape(HT, HEPT))
    du_out = hu_out.reshape(NH, 1)
    di_in = hi_in.reshape(NH, 1)
    di_out = hi_out.reshape(NH, 1)
    du_in = hu_in.reshape(NH, 1)

    srg = er_src.reshape(NS, NB, BE)
    drg = er_dst.reshape(NS, NB, BE)
    srb = eb_src.reshape(NS, NB, BE)
    drb = eb_dst.reshape(NS, NB, BE)

    hu = _tc_project(x_user, W_in_user, b_in_user, du_out)
    hi = _tc_project(x_item, W_in_item, b_in_item, di_out)

    ai1 = _sc_aggregate(*hu, srg, drg)
    au1 = _sc_aggregate(*hi, srb, drb)

    h1i = _tc_mid(*ai1, di_in, W1_rates, b1_rates, di_out)
    h1u = _tc_mid(*au1, du_in, W1_rated_by, b1_rated_by, du_out)

    ai2 = _sc_aggregate(*h1u, srg, drg)
    au2 = _sc_aggregate(*h1i, srb, drb)

    out = _tc_final(*au2, du_in, W2_rated_by, b2_rated_by, half=0)
    out = _tc_final(*ai2, di_in, W2_rates, b2_rates, half=1, prev=out)
    return out


# speedup vs baseline: 1.4829x; 1.0289x over previous
import functools

import jax
import jax.numpy as jnp
from jax import lax
from jax.experimental import pallas as pl
from jax.experimental.pallas import tpu as pltpu
from jax.experimental.pallas import tpu_sc as plsc

N = 10000
E = 80000
D = 512
NCH = 4
CW = D // NCH
NS = 16
EPT = E // NS
BE = 125
NB = EPT // BE
NA = 10240
RPT = NA // NS
ZR = 64

HT = 8
HEPT = E // HT
NH = 10240
HS = NH // NS

_f32 = jnp.float32
_bf16 = jnp.bfloat16
_i32 = jnp.int32


def _sc_mesh():
    return plsc.VectorSubcoreMesh(core_axis_name="c", subcore_axis_name="s",
                                  num_cores=2, num_subcores=NS)


_SC_PARAMS = pltpu.CompilerParams(needs_layout_passes=False)


def _sc_degrees(ia, ib, ic, id_):
    out_t = tuple(jax.ShapeDtypeStruct((NH,), _f32) for _ in range(4))

    @functools.partial(
        pl.kernel, out_type=out_t, mesh=_sc_mesh(),
        compiler_params=_SC_PARAMS,
        scratch_types=dict(
            idxv=pltpu.VMEM((HEPT,), _i32),
            hist=pltpu.VMEM((NH,), _f32),
            red=pltpu.VMEM((HT, HS), _f32),
            res=pltpu.VMEM((HS,), _f32),
            shist=pltpu.VMEM_SHARED((2, HT, NH), _f32),
        ),
    )
    def k(a_hbm, b_hbm, c_hbm, d_hbm, oa, ob, oc, od,
          idxv, hist, red, res, shist):
        c = lax.axis_index("c")
        s = lax.axis_index("s")
        grp = s // HT
        row = s - grp * HT

        for cc, g, ref in ((0, 0, a_hbm), (0, 1, b_hbm),
                           (1, 0, c_hbm), (1, 1, d_hbm)):
            @pl.when(jnp.logical_and(c == cc, grp == g))
            def _(ref=ref):
                pltpu.sync_copy(ref.at[row], idxv)

        zero16 = jnp.zeros((16,), _f32)

        @pl.loop(0, NH, step=16)
        def _(i):
            hist[pl.ds(i, 16)] = zero16

        ones16 = jnp.ones((16,), _f32)

        @pl.loop(0, HEPT, step=16)
        def _(i):
            idx16 = idxv[pl.ds(i, 16)]
            plsc.addupdate_scatter(hist, [idx16], ones16)

        pltpu.sync_copy(hist, shist.at[grp, row])
        plsc.subcore_barrier()

        for g in range(2):
            pltpu.sync_copy(shist.at[g, :, pl.ds(s * HS, HS)], red)

            @pl.loop(0, HS, step=16)
            def _(i):
                acc = red[0, pl.ds(i, 16)]
                for r in range(1, HT):
                    acc = acc + red[r, pl.ds(i, 16)]
                res[pl.ds(i, 16)] = acc

            for cc, out in ((0, (oa, ob)[g]), (1, (oc, od)[g])):
                @pl.when(c == cc)
                def _(out=out):
                    pltpu.sync_copy(res, out.at[pl.ds(s * HS, HS)])

    return k(ia, ib, ic, id_)


def _edge_pass(h, srcv, dstv, g0, g1, acc, sem0, sem1):
    pltpu.async_copy(h.at[srcv.at[0]], g0, sem0)

    @pl.loop(0, NB, step=2)
    def _(j):
        pltpu.async_copy(h.at[srcv.at[j + 1]], g1, sem1)
        pltpu.make_async_copy(h.at[srcv.at[j]], g0, sem0).wait()
        pltpu.sync_copy(g0, acc.at[dstv.at[j]], add=True)

        @pl.when(j + 2 < NB)
        def _():
            pltpu.async_copy(h.at[srcv.at[j + 2]], g0, sem0)

        pltpu.make_async_copy(h.at[srcv.at[j + 1]], g1, sem1).wait()
        pltpu.sync_copy(g1, acc.at[dstv.at[j + 1]], add=True)


def _sc_aggregate(h0, h1, h2, h3, src_g, dst_g):
    out_t = tuple(jax.ShapeDtypeStruct((NA, CW), _f32) for _ in range(4))

    @functools.partial(
        pl.kernel, out_type=out_t, mesh=_sc_mesh(),
        compiler_params=_SC_PARAMS,
        scratch_types=dict(
            srcv=pltpu.VMEM((NB, BE), _i32),
            dstv=pltpu.VMEM((NB, BE), _i32),
            g0=pltpu.VMEM((BE, CW), _f32),
            g1=pltpu.VMEM((BE, CW), _f32),
            acc=pltpu.VMEM_SHARED((NA, CW), _f32),
            sem0=pltpu.SemaphoreType.DMA,
            sem1=pltpu.SemaphoreType.DMA,
        ),
    )
    def k(h0_hbm, h1_hbm, h2_hbm, h3_hbm, src_hbm, dst_hbm,
          o0, o1, o2, o3, srcv, dstv, g0, g1, acc, sem0, sem1):
        c = lax.axis_index("c")
        s = lax.axis_index("s")
        pltpu.sync_copy(src_hbm.at[s], srcv)
        pltpu.sync_copy(dst_hbm.at[s], dstv)

        zero16 = jnp.zeros((16,), _f32)

        hs = (h0_hbm, h1_hbm, h2_hbm, h3_hbm)
        outs = (o0, o1, o2, o3)
        for it in range(2):
            @pl.loop(0, ZR)
            def _(i):
                for j in range(CW // 16):
                    g0[i, pl.ds(j * 16, 16)] = zero16

            for z in range(RPT // ZR):
                pltpu.sync_copy(g0.at[pl.ds(0, ZR)],
                                acc.at[pl.ds(s * RPT + z * ZR, ZR)])
            plsc.subcore_barrier()
            for core in range(2):
                @pl.when(c == core)
                def _(h=hs[core * 2 + it]):
                    _edge_pass(h, srcv, dstv, g0, g1, acc, sem0, sem1)
            plsc.subcore_barrier()
            for core in range(2):
                @pl.when(c == core)
                def _(o=outs[core * 2 + it]):
                    pltpu.sync_copy(acc.at[pl.ds(s * RPT, RPT)],
                                    o.at[pl.ds(s * RPT, RPT)])
            plsc.subcore_barrier()

    return k(h0, h1, h2, h3, src_g, dst_g)


_BN = 1000


def _rsqrt_clip(d):
    return lax.rsqrt(jnp.maximum(d, 1.0))


def _tc_project(x, W, b, deg_out):
    def body(x_ref, w_ref, b_ref, d_ref, *o_refs):
        y = jnp.dot(x_ref[...].astype(_bf16), w_ref[...].astype(_bf16),
                    preferred_element_type=_f32)
        y = (y + b_ref[...]) * _rsqrt_clip(d_ref[...])
        for ci in range(NCH):
            o_refs[ci][...] = y[:, ci * CW:(ci + 1) * CW]

    return pl.pallas_call(
        body,
        grid=(N // _BN,),
        in_specs=[
            pl.BlockSpec((_BN, D), lambda i: (i, 0)),
            pl.BlockSpec((D, D), lambda i: (0, 0)),
            pl.BlockSpec((1, D), lambda i: (0, 0)),
            pl.BlockSpec((_BN, 1), lambda i: (i, 0)),
        ],
        out_specs=[pl.BlockSpec((_BN, CW), lambda i: (i, 0))] * NCH,
        out_shape=[jax.ShapeDtypeStruct((N, CW), _f32)] * NCH,
    )(x, W, b.reshape(1, D), deg_out)


def _tc_mid(a0, a1, a2, a3, deg_in, W, b, deg_out):
    def body(a0r, a1r, a2r, a3r, di_r, w_ref, b_ref, do_r, *o_refs):
        x = jnp.concatenate([a0r[...], a1r[...], a2r[...], a3r[...]], axis=1)
        x = (x * _rsqrt_clip(di_r[...])).astype(_bf16)
        y = (jnp.dot(x, w_ref[...].astype(_bf16), preferred_element_type=_f32)
             + b_ref[...])
        y = jnp.maximum(y, 0.0) * _rsqrt_clip(do_r[...])
        for ci in range(NCH):
            o_refs[ci][...] = y[:, ci * CW:(ci + 1) * CW]

    return pl.pallas_call(
        body,
        grid=(N // _BN,),
        in_specs=[pl.BlockSpec((_BN, CW), lambda i: (i, 0))] * NCH + [
            pl.BlockSpec((_BN, 1), lambda i: (i, 0)),
            pl.BlockSpec((D, D), lambda i: (0, 0)),
            pl.BlockSpec((1, D), lambda i: (0, 0)),
            pl.BlockSpec((_BN, 1), lambda i: (i, 0)),
        ],
        out_specs=[pl.BlockSpec((_BN, CW), lambda i: (i, 0))] * NCH,
        out_shape=[jax.ShapeDtypeStruct((N, CW), _f32)] * NCH,
    )(a0, a1, a2, a3, deg_in, W, b.reshape(1, D), deg_out)


def _tc_final(a0, a1, a2, a3, deg_in, W, b, half, prev=None):
    def body(a0r, a1r, a2r, a3r, di_r, w_ref, b_ref, *refs):
        o_ref = refs[-1]
        x = jnp.concatenate([a0r[...], a1r[...], a2r[...], a3r[...]], axis=1)
        x = (x * _rsqrt_clip(di_r[...])).astype(_bf16)
        o_ref[...] = (jnp.dot(x, w_ref[...].astype(_bf16),
                              preferred_element_type=_f32) + b_ref[...])

    nb = N // _BN
    in_specs = [pl.BlockSpec((_BN, CW), lambda i: (i, 0))] * NCH + [
        pl.BlockSpec((_BN, 1), lambda i: (i, 0)),
        pl.BlockSpec((D, D), lambda i: (0, 0)),
        pl.BlockSpec((1, D), lambda i: (0, 0)),
    ]
    args = [a0, a1, a2, a3, deg_in, W, b.reshape(1, D)]
    aliases = {}
    if prev is not None:
        in_specs = in_specs + [pl.BlockSpec(memory_space=pl.ANY)]
        args.append(prev)
        aliases = {len(args) - 1: 0}
    return pl.pallas_call(
        body,
        grid=(nb,),
        in_specs=in_specs,
        out_specs=pl.BlockSpec((_BN, D), lambda i: (i + half * nb, 0)),
        out_shape=jax.ShapeDtypeStruct((2 * N, D), _f32),
        input_output_aliases=aliases,
    )(*args)


def kernel(x_user, x_item, edge_rates, edge_rated_by,
           W_in_user, b_in_user, W_in_item, b_in_item,
           W1_rates, b1_rates, W1_rated_by, b1_rated_by,
           W2_rates, b2_rates, W2_rated_by, b2_rated_by):
    er_src = edge_rates[0].astype(_i32)
    er_dst = edge_rates[1].astype(_i32)
    eb_src = edge_rated_by[0].astype(_i32)
    eb_dst = edge_rated_by[1].astype(_i32)

    hu_out, hi_in, hi_out, hu_in = _sc_degrees(
        er_src.reshape(HT, HEPT), er_dst.reshape(HT, HEPT),
        eb_src.reshape(HT, HEPT), eb_dst.reshape(HT, HEPT))
    du_out = hu_out.reshape(NH, 1)
    di_in = hi_in.reshape(NH, 1)
    di_out = hi_out.reshape(NH, 1)
    du_in = hu_in.reshape(NH, 1)

    srg = er_src.reshape(NS, NB, BE)
    drg = er_dst.reshape(NS, NB, BE)
    srb = eb_src.reshape(NS, NB, BE)
    drb = eb_dst.reshape(NS, NB, BE)

    hu = _tc_project(x_user, W_in_user, b_in_user, du_out)
    hi = _tc_project(x_item, W_in_item, b_in_item, di_out)

    ai1 = _sc_aggregate(*hu, srg, drg)
    au1 = _sc_aggregate(*hi, srb, drb)

    h1i = _tc_mid(*ai1, di_in, W1_rates, b1_rates, di_out)
    h1u = _tc_mid(*au1, du_in, W1_rated_by, b1_rated_by, du_out)

    ai2 = _sc_aggregate(*h1u, srg, drg)
    au2 = _sc_aggregate(*h1i, srb, drb)

    out = _tc_final(*au2, du_in, W2_rated_by, b2_rated_by, half=0)
    out = _tc_final(*ai2, di_in, W2_rates, b2_rates, half=1, prev=out)
    return out


# speedup vs baseline: 1.8967x; 1.2791x over previous
import functools

import jax
import jax.numpy as jnp
from jax import lax
from jax.experimental import pallas as pl
from jax.experimental.pallas import tpu as pltpu
from jax.experimental.pallas import tpu_sc as plsc

N = 10000
E = 80000
D = 512
NCH = 2
CW = D // NCH
NS = 16
EPT = E // NS
BE = 125
NB = EPT // BE
NA = 10240
RPT = NA // NS
ZR = 64

HT = 8
HEPT = E // HT
NH = 10240
HS = NH // NS

_f32 = jnp.float32
_bf16 = jnp.bfloat16
_i32 = jnp.int32


def _sc_mesh():
    return plsc.VectorSubcoreMesh(core_axis_name="c", subcore_axis_name="s",
                                  num_cores=2, num_subcores=NS)


_SC_PARAMS = pltpu.CompilerParams(needs_layout_passes=False,
                                  use_tc_tiling_on_sc=False)


def _sc_degrees(ia, ib, ic, id_):
    out_t = tuple(jax.ShapeDtypeStruct((NH,), _f32) for _ in range(4))

    @functools.partial(
        pl.kernel, out_type=out_t, mesh=_sc_mesh(),
        compiler_params=_SC_PARAMS,
        scratch_types=dict(
            idxv=pltpu.VMEM((HEPT,), _i32),
            hist=pltpu.VMEM((NH,), _f32),
            red=pltpu.VMEM((HT, HS), _f32),
            res=pltpu.VMEM((HS,), _f32),
            shist=pltpu.VMEM_SHARED((2, HT, NH), _f32),
        ),
    )
    def k(a_hbm, b_hbm, c_hbm, d_hbm, oa, ob, oc, od,
          idxv, hist, red, res, shist):
        c = lax.axis_index("c")
        s = lax.axis_index("s")
        grp = s // HT
        row = s - grp * HT

        for cc, g, ref in ((0, 0, a_hbm), (0, 1, b_hbm),
                           (1, 0, c_hbm), (1, 1, d_hbm)):
            @pl.when(jnp.logical_and(c == cc, grp == g))
            def _(ref=ref):
                pltpu.sync_copy(ref.at[row], idxv)

        zero16 = jnp.zeros((16,), _f32)

        @pl.loop(0, NH, step=16)
        def _(i):
            hist[pl.ds(i, 16)] = zero16

        ones16 = jnp.ones((16,), _f32)

        @pl.loop(0, HEPT, step=16)
        def _(i):
            idx16 = idxv[pl.ds(i, 16)]
            plsc.addupdate_scatter(hist, [idx16], ones16)

        pltpu.sync_copy(hist, shist.at[grp, row])
        plsc.subcore_barrier()

        for g in range(2):
            pltpu.sync_copy(shist.at[g, :, pl.ds(s * HS, HS)], red)

            @pl.loop(0, HS, step=16)
            def _(i):
                acc = red[0, pl.ds(i, 16)]
                for r in range(1, HT):
                    acc = acc + red[r, pl.ds(i, 16)]
                res[pl.ds(i, 16)] = acc

            for cc, out in ((0, (oa, ob)[g]), (1, (oc, od)[g])):
                @pl.when(c == cc)
                def _(out=out):
                    pltpu.sync_copy(res, out.at[pl.ds(s * HS, HS)])

    return k(ia, ib, ic, id_)


def _edge_pass(h, srcv, dstv, g0, g1, acc, sem0, sem1):
    pltpu.async_copy(h.at[srcv.at[0]], g0, sem0)

    @pl.loop(0, NB, step=2)
    def _(j):
        pltpu.async_copy(h.at[srcv.at[j + 1]], g1, sem1)
        pltpu.make_async_copy(h.at[srcv.at[j]], g0, sem0).wait()
        pltpu.sync_copy(g0, acc.at[dstv.at[j]], add=True)

        @pl.when(j + 2 < NB)
        def _():
            pltpu.async_copy(h.at[srcv.at[j + 2]], g0, sem0)

        pltpu.make_async_copy(h.at[srcv.at[j + 1]], g1, sem1).wait()
        pltpu.sync_copy(g1, acc.at[dstv.at[j + 1]], add=True)


def _sc_aggregate(h0, h1, src_g, dst_g):
    out_t = tuple(jax.ShapeDtypeStruct((NA, CW), _bf16) for _ in range(2))

    @functools.partial(
        pl.kernel, out_type=out_t, mesh=_sc_mesh(),
        compiler_params=_SC_PARAMS,
        scratch_types=dict(
            srcv=pltpu.VMEM((NB, BE), _i32),
            dstv=pltpu.VMEM((NB, BE), _i32),
            g0=pltpu.VMEM((BE, CW), _bf16),
            g1=pltpu.VMEM((BE, CW), _bf16),
            acc=pltpu.VMEM_SHARED((NA, CW), _bf16),
            sem0=pltpu.SemaphoreType.DMA,
            sem1=pltpu.SemaphoreType.DMA,
        ),
    )
    def k(h0_hbm, h1_hbm, src_hbm, dst_hbm, o0, o1,
          srcv, dstv, g0, g1, acc, sem0, sem1):
        c = lax.axis_index("c")
        s = lax.axis_index("s")
        pltpu.sync_copy(src_hbm.at[s], srcv)
        pltpu.sync_copy(dst_hbm.at[s], dstv)

        zero32 = jnp.zeros((32,), _bf16)

        @pl.loop(0, ZR)
        def _(i):
            for j in range(CW // 32):
                g0[i, pl.ds(j * 32, 32)] = zero32

        for z in range(RPT // ZR):
            pltpu.sync_copy(g0.at[pl.ds(0, ZR)],
                            acc.at[pl.ds(s * RPT + z * ZR, ZR)])
        plsc.subcore_barrier()
        for core, h in ((0, h0_hbm), (1, h1_hbm)):
            @pl.when(c == core)
            def _(h=h):
                _edge_pass(h, srcv, dstv, g0, g1, acc, sem0, sem1)
        plsc.subcore_barrier()
        for core, o in ((0, o0), (1, o1)):
            @pl.when(c == core)
            def _(o=o):
                pltpu.sync_copy(acc.at[pl.ds(s * RPT, RPT)],
                                o.at[pl.ds(s * RPT, RPT)])

    return k(h0, h1, src_g, dst_g)


_BN = 2000


def _rsqrt_clip(d):
    return lax.rsqrt(jnp.maximum(d, 1.0))


def _tc_project(x, W, b, deg_out):
    def body(x_ref, w_ref, b_ref, d_ref, *o_refs):
        y = jnp.dot(x_ref[...].astype(_bf16), w_ref[...].astype(_bf16),
                    preferred_element_type=_f32)
        y = (y + b_ref[...]) * _rsqrt_clip(d_ref[...])
        for ci in range(NCH):
            o_refs[ci][...] = y[:, ci * CW:(ci + 1) * CW].astype(_bf16)

    return pl.pallas_call(
        body,
        grid=(N // _BN,),
        in_specs=[
            pl.BlockSpec((_BN, D), lambda i: (i, 0)),
            pl.BlockSpec((D, D), lambda i: (0, 0)),
            pl.BlockSpec((1, D), lambda i: (0, 0)),
            pl.BlockSpec((_BN, 1), lambda i: (i, 0)),
        ],
        out_specs=[pl.BlockSpec((_BN, CW), lambda i: (i, 0))] * NCH,
        out_shape=[jax.ShapeDtypeStruct((N, CW), _bf16)] * NCH,
    )(x, W, b.reshape(1, D), deg_out)


def _tc_mid(a0, a1, deg_in, W, b, deg_out):
    def body(a0r, a1r, di_r, w_ref, b_ref, do_r, *o_refs):
        x = jnp.concatenate([a0r[...], a1r[...]], axis=1).astype(_f32)
        x = (x * _rsqrt_clip(di_r[...])).astype(_bf16)
        y = (jnp.dot(x, w_ref[...].astype(_bf16), preferred_element_type=_f32)
             + b_ref[...])
        y = jnp.maximum(y, 0.0) * _rsqrt_clip(do_r[...])
        for ci in range(NCH):
            o_refs[ci][...] = y[:, ci * CW:(ci + 1) * CW].astype(_bf16)

    return pl.pallas_call(
        body,
        grid=(N // _BN,),
        in_specs=[pl.BlockSpec((_BN, CW), lambda i: (i, 0))] * NCH + [
            pl.BlockSpec((_BN, 1), lambda i: (i, 0)),
            pl.BlockSpec((D, D), lambda i: (0, 0)),
            pl.BlockSpec((1, D), lambda i: (0, 0)),
            pl.BlockSpec((_BN, 1), lambda i: (i, 0)),
        ],
        out_specs=[pl.BlockSpec((_BN, CW), lambda i: (i, 0))] * NCH,
        out_shape=[jax.ShapeDtypeStruct((NA, CW), _bf16)] * NCH,
    )(a0, a1, deg_in, W, b.reshape(1, D), deg_out)


def _tc_final(a0, a1, deg_in, W, b, half, prev=None):
    def body(a0r, a1r, di_r, w_ref, b_ref, *refs):
        o_ref = refs[-1]
        x = jnp.concatenate([a0r[...], a1r[...]], axis=1).astype(_f32)
        x = (x * _rsqrt_clip(di_r[...])).astype(_bf16)
        o_ref[...] = (jnp.dot(x, w_ref[...].astype(_bf16),
                              preferred_element_type=_f32) + b_ref[...])

    nb = N // _BN
    in_specs = [pl.BlockSpec((_BN, CW), lambda i: (i, 0))] * NCH + [
        pl.BlockSpec((_BN, 1), lambda i: (i, 0)),
        pl.BlockSpec((D, D), lambda i: (0, 0)),
        pl.BlockSpec((1, D), lambda i: (0, 0)),
    ]
    args = [a0, a1, deg_in, W, b.reshape(1, D)]
    aliases = {}
    if prev is not None:
        in_specs = in_specs + [pl.BlockSpec(memory_space=pl.ANY)]
        args.append(prev)
        aliases = {len(args) - 1: 0}
    return pl.pallas_call(
        body,
        grid=(nb,),
        in_specs=in_specs,
        out_specs=pl.BlockSpec((_BN, D), lambda i: (i + half * nb, 0)),
        out_shape=jax.ShapeDtypeStruct((2 * N, D), _f32),
        input_output_aliases=aliases,
    )(*args)


def kernel(x_user, x_item, edge_rates, edge_rated_by,
           W_in_user, b_in_user, W_in_item, b_in_item,
           W1_rates, b1_rates, W1_rated_by, b1_rated_by,
           W2_rates, b2_rates, W2_rated_by, b2_rated_by):
    er_src = edge_rates[0].astype(_i32)
    er_dst = edge_rates[1].astype(_i32)
    eb_src = edge_rated_by[0].astype(_i32)
    eb_dst = edge_rated_by[1].astype(_i32)

    hu_out, hi_in, hi_out, hu_in = _sc_degrees(
        er_src.reshape(HT, HEPT), er_dst.reshape(HT, HEPT),
        eb_src.reshape(HT, HEPT), eb_dst.reshape(HT, HEPT))
    du_out = hu_out.reshape(NH, 1)
    di_in = hi_in.reshape(NH, 1)
    di_out = hi_out.reshape(NH, 1)
    du_in = hu_in.reshape(NH, 1)

    srg = er_src.reshape(NS, NB, BE)
    drg = er_dst.reshape(NS, NB, BE)
    srb = eb_src.reshape(NS, NB, BE)
    drb = eb_dst.reshape(NS, NB, BE)

    hu = _tc_project(x_user, W_in_user, b_in_user, du_out)
    hi = _tc_project(x_item, W_in_item, b_in_item, di_out)

    ai1 = _sc_aggregate(*hu, srg, drg)
    au1 = _sc_aggregate(*hi, srb, drb)

    h1i = _tc_mid(*ai1, di_in, W1_rates, b1_rates, di_out)
    h1u = _tc_mid(*au1, du_in, W1_rated_by, b1_rated_by, du_out)

    ai2 = _sc_aggregate(*h1u, srg, drg)
    au2 = _sc_aggregate(*h1i, srb, drb)

    out = _tc_final(*au2, du_in, W2_rated_by, b2_rated_by, half=0)
    out = _tc_final(*ai2, di_in, W2_rates, b2_rates, half=1, prev=out)
    return out
